# baseline (speedup 1.0000x reference)
import functools

import jax
import jax.numpy as jnp
from jax import lax
from jax.experimental import pallas as pl
from jax.experimental.pallas import tpu as pltpu
from jax.experimental.pallas import tpu_sc as plsc

_B = 16384
_D = 32
_NC, _NS = 2, 16
_NW = _NC * _NS
_BPW = _B // _NW
_CH = 128
_NCH = _BPW // _CH

@functools.lru_cache(maxsize=None)
def _gather_pairs_kernel():
    mesh = plsc.VectorSubcoreMesh(core_axis_name="c", subcore_axis_name="s",
                                  num_cores=_NC, num_subcores=_NS)

    @functools.partial(
        pl.kernel,
        mesh=mesh,
        out_type=(
            jax.ShapeDtypeStruct((_B, _D), jnp.float32),
            jax.ShapeDtypeStruct((_B, _D), jnp.float32),
        ),
        scratch_types=[
            pltpu.VMEM((_NCH, _CH), jnp.int32),
            pltpu.VMEM((_NCH, _CH), jnp.int32),
            pltpu.VMEM((_BPW, _D), jnp.float32),
            pltpu.VMEM((_BPW, _D), jnp.float32),
            pltpu.SemaphoreType.DMA,
            pltpu.SemaphoreType.DMA,
        ],
        compiler_params=pltpu.CompilerParams(use_tc_tiling_on_sc=False),
    )
    def _gather_pairs(uidx_hbm, iidx_hbm, utab_hbm, itab_hbm,
                      uout_hbm, iout_hbm,
                      uidx_v, iidx_v, urows_v, irows_v, usem, isem):
        wid = lax.axis_index("s") * _NC + lax.axis_index("c")
        pltpu.sync_copy(uidx_hbm.at[pl.ds(wid * _NCH, _NCH)], uidx_v)
        pltpu.sync_copy(iidx_hbm.at[pl.ds(wid * _NCH, _NCH)], iidx_v)
        copies = []
        for j in range(_NCH):
            copies.append(pltpu.async_copy(
                utab_hbm.at[uidx_v.at[j]], urows_v.at[pl.ds(j * _CH, _CH)],
                usem))
            copies.append(pltpu.async_copy(
                itab_hbm.at[iidx_v.at[j]], irows_v.at[pl.ds(j * _CH, _CH)],
                isem))
        for c in copies:
            c.wait()
        base = wid * _BPW
        pltpu.sync_copy(urows_v, uout_hbm.at[pl.ds(base, _BPW)])
        pltpu.sync_copy(irows_v, iout_hbm.at[pl.ds(base, _BPW)])

    return _gather_pairs


_BM = 2048


def _mlp_body(u_ref, v_ref, w1_ref, b1_ref, w2_ref, b2_ref, w3_ref, b3_ref,
              o_ref):
    x1 = (jnp.dot(u_ref[...], w1_ref[0:_D, :],
                  preferred_element_type=jnp.float32)
          + jnp.dot(v_ref[...], w1_ref[_D:2 * _D, :],
                    preferred_element_type=jnp.float32)
          + b1_ref[...])
    h1 = jnp.maximum(x1, 0.0)
    h2 = jnp.maximum(
        jnp.dot(h1, w2_ref[...], preferred_element_type=jnp.float32)
        + b2_ref[...], 0.0)
    o_ref[...] = (jnp.dot(h2, w3_ref[...], preferred_element_type=jnp.float32)
                  + b3_ref[...])


def _mlp(u_vec, i_vec, W1, b1, W2, b2, W3, b3):
    return pl.pallas_call(
        _mlp_body,
        grid=(_B // _BM,),
        in_specs=[
            pl.BlockSpec((_BM, _D), lambda m: (m, 0)),
            pl.BlockSpec((_BM, _D), lambda m: (m, 0)),
            pl.BlockSpec((2 * _D, 64), lambda m: (0, 0)),
            pl.BlockSpec((1, 64), lambda m: (0, 0)),
            pl.BlockSpec((64, 32), lambda m: (0, 0)),
            pl.BlockSpec((1, 32), lambda m: (0, 0)),
            pl.BlockSpec((32, 1), lambda m: (0, 0)),
            pl.BlockSpec((1, 1), lambda m: (0, 0)),
        ],
        out_specs=pl.BlockSpec((_BM, 1), lambda m: (m, 0)),
        out_shape=jax.ShapeDtypeStruct((_B, 1), jnp.float32),
    )(u_vec, i_vec, W1, b1.reshape(1, 64), W2, b2.reshape(1, 32),
      W3, b3.reshape(1, 1))


def kernel(inputs, user_table, item_table, W1, b1, W2, b2, W3, b3):
    idx = inputs.astype(jnp.int32)
    uidx = idx[:, 0].reshape(_NW * _NCH, _CH)
    iidx = idx[:, 1].reshape(_NW * _NCH, _CH)
    u_vec, i_vec = _gather_pairs_kernel()(uidx, iidx, user_table, item_table)
    return _mlp(u_vec, i_vec, W1, b1, W2, b2, W3, b3)
